# Initial kernel scaffold; baseline (speedup 1.0000x reference)
import jax
import jax.numpy as jnp
from jax.experimental import pallas as pl


def kernel(x, edge_index, batch, edge_attr, params):
    raise NotImplementedError("write your pallas kernel here")


# speedup vs baseline: 2.5435x; 2.5435x over previous
import functools

import jax
import jax.numpy as jnp
from jax import lax
from jax.experimental import pallas as pl
from jax.experimental.pallas import tpu as pltpu
from jax.experimental.pallas import tpu_sc as plsc

_N = 10000
_E = 320000
_H = 128
_G = 64
_L = 4

_NC = 2
_NS = 16
_NW = _NC * _NS
_EW = _E // _NW
_K = 80
_NCH = _EW // _K
_CZ = 200
_NQ = _N // _CZ


def _edge_linear(edge_attr, Wp, bp):
    BE = 2000

    def body(x_ref, w_ref, b_ref, o_ref):
        x = x_ref[...]
        for l in range(_L):
            o_ref[l] = jnp.dot(x, w_ref[l], preferred_element_type=jnp.float32) + b_ref[l]

    return pl.pallas_call(
        body,
        grid=(_E // BE,),
        in_specs=[
            pl.BlockSpec((BE, 16), lambda i: (i, 0)),
            pl.BlockSpec((_L, 16, _H), lambda i: (0, 0, 0)),
            pl.BlockSpec((_L, 1, _H), lambda i: (0, 0, 0)),
        ],
        out_specs=pl.BlockSpec((_L, BE, _H), lambda i: (0, i, 0)),
        out_shape=jax.ShapeDtypeStruct((_L, _E, _H), jnp.float32),
    )(edge_attr, Wp, bp)


def _node_embed(x, W, b):
    def body(x_ref, w_ref, b_ref, o_ref):
        o_ref[...] = jnp.dot(x_ref[...], w_ref[...], preferred_element_type=jnp.float32) + b_ref[...]

    return pl.pallas_call(
        body, out_shape=jax.ShapeDtypeStruct((_N, _H), jnp.float32)
    )(x, W, b)


def _sc_edge_pass(l):
    mesh = plsc.VectorSubcoreMesh(core_axis_name="c", subcore_axis_name="s")

    @functools.partial(
        pl.kernel,
        out_type=jax.ShapeDtypeStruct((_NC, _N, _H), jnp.float32),
        mesh=mesh,
        scratch_types=[
            pltpu.VMEM((2, _K), jnp.int32),
            pltpu.VMEM((_K, _H), jnp.float32),
            pltpu.VMEM((_K, _H), jnp.float32),
            pltpu.VMEM((_CZ, _H), jnp.float32),
            pltpu.VMEM_SHARED((_N, _H), jnp.float32),
            pltpu.SemaphoreType.DMA,
        ],
    )
    def k(h_hbm, ea_hbm, src_hbm, dst_hbm, out_hbm, idx_v, hbuf, eabuf, zbuf, agg_sh, sem):
        c = lax.axis_index("c")
        s = lax.axis_index("s")

        zero16 = jnp.zeros((16,), jnp.float32)

        def zrow(e, carry):
            for f in range(_H // 16):
                zbuf[e, pl.ds(f * 16, 16)] = zero16
            return carry

        lax.fori_loop(0, _CZ, zrow, 0)
        nq = (_NQ - s + _NS - 1) // _NS

        def zchunk(t, carry):
            r0 = pl.multiple_of((s + t * _NS) * _CZ, 8)
            pltpu.sync_copy(zbuf, agg_sh.at[pl.ds(r0, _CZ)])
            return carry

        lax.fori_loop(0, nq, zchunk, 0)
        plsc.subcore_barrier()

        w = c * _NS + s

        def chunk(j, carry):
            e0 = pl.multiple_of(w * _EW + j * _K, 8)
            pltpu.sync_copy(src_hbm.at[pl.ds(e0, _K)], idx_v.at[0])
            pltpu.sync_copy(dst_hbm.at[pl.ds(e0, _K)], idx_v.at[1])
            pltpu.async_copy(h_hbm.at[idx_v.at[0]], hbuf, sem).wait()
            pltpu.sync_copy(ea_hbm.at[l, pl.ds(e0, _K)], eabuf)

            def row(e, cc):
                for f in range(_H // 16):
                    sl = pl.ds(f * 16, 16)
                    hbuf[e, sl] = jnp.maximum(hbuf[e, sl] + eabuf[e, sl], 0.0)
                return cc

            lax.fori_loop(0, _K, row, 0)
            pltpu.sync_copy(hbuf, agg_sh.at[idx_v.at[1]], add=True)
            return carry

        lax.fori_loop(0, _NCH, chunk, 0)
        plsc.subcore_barrier()

        def ochunk(t, carry):
            r0 = pl.multiple_of((s + t * _NS) * _CZ, 8)
            pltpu.sync_copy(agg_sh.at[pl.ds(r0, _CZ)], zbuf)
            pltpu.sync_copy(zbuf, out_hbm.at[c, pl.ds(r0, _CZ)])
            return carry

        lax.fori_loop(0, nq, ochunk, 0)

    return k


def _node_update(h, aggp, lp):
    W1, b1 = lp["mlp1"]
    W2, b2 = lp["mlp2"]
    gamma, beta = lp["bn"]

    def body(h_ref, a_ref, w1_ref, b1_ref, w2_ref, b2_ref, g_ref, bb_ref, o_ref):
        z = h_ref[...] + a_ref[0] + a_ref[1]
        a1 = jnp.maximum(
            jnp.dot(z, w1_ref[...], preferred_element_type=jnp.float32) + b1_ref[...], 0.0
        )
        out = jnp.dot(a1, w2_ref[...], preferred_element_type=jnp.float32) + b2_ref[...]
        mean = jnp.mean(out, axis=0, keepdims=True)
        d = out - mean
        var = jnp.mean(d * d, axis=0, keepdims=True)
        o_ref[...] = jnp.maximum(
            d * lax.rsqrt(var + 1e-5) * g_ref[...] + bb_ref[...], 0.0
        )

    return pl.pallas_call(
        body, out_shape=jax.ShapeDtypeStruct((_N, _H), jnp.float32)
    )(
        h, aggp, W1, b1.reshape(1, _H), W2, b2.reshape(1, _H),
        gamma.reshape(1, _H), beta.reshape(1, _H),
    )


def _readout_head(h, batch2d, p1, p2):
    W1, b1 = p1
    W2, b2 = p2

    def body(h_ref, b_ref, w1_ref, b1_ref, w2_ref, b2_ref, o_ref):
        oh = (b_ref[...] == lax.broadcasted_iota(jnp.int32, (_G, _N), 0)).astype(jnp.float32)
        g = jnp.dot(oh, h_ref[...], preferred_element_type=jnp.float32)
        a = jnp.maximum(
            jnp.dot(g, w1_ref[...], preferred_element_type=jnp.float32) + b1_ref[...], 0.0
        )
        o_ref[...] = jnp.dot(a, w2_ref[...], preferred_element_type=jnp.float32) + b2_ref[...]

    return pl.pallas_call(
        body, out_shape=jax.ShapeDtypeStruct((_G, 12), jnp.float32)
    )(h, batch2d, W1, b1.reshape(1, -1), W2, b2.reshape(1, -1))


def kernel(x, edge_index, batch, edge_attr, params):
    We, be = params["edge_emb"]
    Wn, bn = params["node_emb"]
    Wp = jnp.stack([We @ lp["lin_e"][0] for lp in params["layers"]])
    bp = jnp.stack(
        [(be @ lp["lin_e"][0] + lp["lin_e"][1]).reshape(1, _H) for lp in params["layers"]]
    )
    ea_all = _edge_linear(edge_attr, Wp, bp)
    src = edge_index[0]
    dst = edge_index[1]
    h = _node_embed(x, Wn, bn.reshape(1, _H))
    for l, lp in enumerate(params["layers"]):
        aggp = _sc_edge_pass(l)(h, ea_all, src, dst)
        h = _node_update(h, aggp, lp)
    return _readout_head(h, batch.reshape(1, _N), params["head1"], params["head2"])


# speedup vs baseline: 4.5784x; 1.8000x over previous
import functools

import jax
import jax.numpy as jnp
from jax import lax
from jax.experimental import pallas as pl
from jax.experimental.pallas import tpu as pltpu
from jax.experimental.pallas import tpu_sc as plsc

_N = 10000
_E = 320000
_H = 128
_G = 64
_L = 4

_NC = 2
_NS = 16
_NW = _NC * _NS
_EW = _E // _NW
_K = 80
_NCH = _EW // _K
_CZ = 80
_NQ = _N // _CZ
_NIB = 4


def _edge_linear(edge_attr, Wp, bp):
    BE = 2000

    def body(x_ref, w_ref, b_ref, o_ref):
        x = x_ref[...]
        for l in range(_L):
            o_ref[l] = jnp.dot(x, w_ref[l], preferred_element_type=jnp.float32) + b_ref[l]

    return pl.pallas_call(
        body,
        grid=(_E // BE,),
        in_specs=[
            pl.BlockSpec((BE, 16), lambda i: (i, 0)),
            pl.BlockSpec((_L, 16, _H), lambda i: (0, 0, 0)),
            pl.BlockSpec((_L, 1, _H), lambda i: (0, 0, 0)),
        ],
        out_specs=pl.BlockSpec((_L, BE, _H), lambda i: (0, i, 0)),
        out_shape=jax.ShapeDtypeStruct((_L, _E, _H), jnp.float32),
    )(edge_attr, Wp, bp)


def _node_embed(x, W, b):
    def body(x_ref, w_ref, b_ref, o_ref):
        o_ref[...] = jnp.dot(x_ref[...], w_ref[...], preferred_element_type=jnp.float32) + b_ref[...]

    return pl.pallas_call(
        body, out_shape=jax.ShapeDtypeStruct((_N, _H), jnp.float32)
    )(x, W, b)


def _sc_edge_pass(l):
    mesh = plsc.VectorSubcoreMesh(core_axis_name="c", subcore_axis_name="s")

    @functools.partial(
        pl.kernel,
        out_type=jax.ShapeDtypeStruct((_NC, _N, _H), jnp.float32),
        mesh=mesh,
        scratch_types=[
            pltpu.VMEM((2 * _NIB, _K), jnp.int32),
            pltpu.VMEM((2, _K, _H), jnp.float32),
            pltpu.VMEM((2, _K, _H), jnp.float32),
            pltpu.VMEM_SHARED((_N, _H), jnp.float32),
            pltpu.SemaphoreType.DMA((_NIB,)),
            pltpu.SemaphoreType.DMA((2,)),
            pltpu.SemaphoreType.DMA((2,)),
            pltpu.SemaphoreType.DMA((2,)),
        ],
    )
    def k(h_hbm, ea_hbm, src_hbm, dst_hbm, out_hbm, idxb, hbuf, eabuf,
          agg_sh, sem_i, sem_g, sem_e, sem_s):
        c = lax.axis_index("c")
        s = lax.axis_index("s")
        w = c * _NS + s

        zero16 = jnp.zeros((16,), jnp.float32)

        def zrow(e, carry):
            for f in range(_H // 16):
                hbuf[0, e, pl.ds(f * 16, 16)] = zero16
            return carry

        lax.fori_loop(0, _CZ, zrow, 0)
        nq = (_NQ - s + _NS - 1) // _NS

        def zchunk(t, carry):
            r0 = pl.multiple_of((s + t * _NS) * _CZ, 8)
            pltpu.sync_copy(hbuf.at[0], agg_sh.at[pl.ds(r0, _CZ)])
            return carry

        lax.fori_loop(0, nq, zchunk, 0)
        plsc.subcore_barrier()

        def issue_idx(j, m):
            e0 = pl.multiple_of(w * _EW + j * _K, 8)
            pltpu.async_copy(src_hbm.at[pl.ds(e0, _K)], idxb.at[2 * m], sem_i.at[m])
            pltpu.async_copy(dst_hbm.at[pl.ds(e0, _K)], idxb.at[2 * m + 1], sem_i.at[m])

        def wait_idx(j, m):
            e0 = pl.multiple_of(w * _EW + j * _K, 8)
            pltpu.make_async_copy(src_hbm.at[pl.ds(e0, _K)], idxb.at[2 * m], sem_i.at[m]).wait()
            pltpu.make_async_copy(dst_hbm.at[pl.ds(e0, _K)], idxb.at[2 * m + 1], sem_i.at[m]).wait()

        def issue_gather(m, b):
            pltpu.async_copy(h_hbm.at[idxb.at[2 * m]], hbuf.at[b], sem_g.at[b])

        def wait_gather(m, b):
            pltpu.make_async_copy(h_hbm.at[idxb.at[2 * m]], hbuf.at[b], sem_g.at[b]).wait()

        def issue_ea(j, b):
            e0 = pl.multiple_of(w * _EW + j * _K, 8)
            pltpu.async_copy(ea_hbm.at[l, pl.ds(e0, _K)], eabuf.at[b], sem_e.at[b])

        def wait_ea(j, b):
            e0 = pl.multiple_of(w * _EW + j * _K, 8)
            pltpu.make_async_copy(ea_hbm.at[l, pl.ds(e0, _K)], eabuf.at[b], sem_e.at[b]).wait()

        def issue_scatter(m, b):
            pltpu.async_copy(eabuf.at[b], agg_sh.at[idxb.at[2 * m + 1]], sem_s.at[b], add=True)

        def wait_scatter(m, b):
            pltpu.make_async_copy(eabuf.at[b], agg_sh.at[idxb.at[2 * m + 1]], sem_s.at[b]).wait()

        def compute(b):
            @plsc.parallel_loop(0, _K, 1, unroll=2)
            def row(e):
                for f in range(_H // 16):
                    sl = pl.ds(f * 16, 16)
                    eabuf[b, e, sl] = jnp.maximum(hbuf[b, e, sl] + eabuf[b, e, sl], 0.0)

        def body(j, m):
            b = m % 2

            wait_gather(m, b)
            wait_ea(j, b)

            @pl.when(j + 1 < _NCH)
            def _():
                wait_idx(j + 1, (m + 1) % _NIB)
                issue_gather((m + 1) % _NIB, 1 - b)

            @pl.when(j + 2 < _NCH)
            def _():
                issue_idx(j + 2, (m + 2) % _NIB)

            compute(b)

            @pl.when(j >= 1)
            def _():
                wait_scatter((m + _NIB - 1) % _NIB, 1 - b)

            @pl.when(j + 1 < _NCH)
            def _():
                issue_ea(j + 1, 1 - b)

            issue_scatter(m, b)

        issue_idx(0, 0)
        issue_idx(1, 1)
        wait_idx(0, 0)
        issue_gather(0, 0)
        issue_ea(0, 0)

        def quad(q, carry):
            for m in range(_NIB):
                body(_NIB * q + m, m)
            return carry

        lax.fori_loop(0, _NCH // _NIB, quad, 0)
        body(_NCH - 1, 0)
        wait_scatter(0, 0)
        plsc.subcore_barrier()

        def ochunk(t, carry):
            r0 = pl.multiple_of((s + t * _NS) * _CZ, 8)
            pltpu.sync_copy(agg_sh.at[pl.ds(r0, _CZ)], hbuf.at[0])
            pltpu.sync_copy(hbuf.at[0], out_hbm.at[c, pl.ds(r0, _CZ)])
            return carry

        lax.fori_loop(0, nq, ochunk, 0)

    return k


def _node_update(h, aggp, lp):
    W1, b1 = lp["mlp1"]
    W2, b2 = lp["mlp2"]
    gamma, beta = lp["bn"]

    def body(h_ref, a_ref, w1_ref, b1_ref, w2_ref, b2_ref, g_ref, bb_ref, o_ref):
        z = h_ref[...] + a_ref[0] + a_ref[1]
        a1 = jnp.maximum(
            jnp.dot(z, w1_ref[...], preferred_element_type=jnp.float32) + b1_ref[...], 0.0
        )
        out = jnp.dot(a1, w2_ref[...], preferred_element_type=jnp.float32) + b2_ref[...]
        mean = jnp.mean(out, axis=0, keepdims=True)
        d = out - mean
        var = jnp.mean(d * d, axis=0, keepdims=True)
        o_ref[...] = jnp.maximum(
            d * lax.rsqrt(var + 1e-5) * g_ref[...] + bb_ref[...], 0.0
        )

    return pl.pallas_call(
        body, out_shape=jax.ShapeDtypeStruct((_N, _H), jnp.float32)
    )(
        h, aggp, W1, b1.reshape(1, _H), W2, b2.reshape(1, _H),
        gamma.reshape(1, _H), beta.reshape(1, _H),
    )


def _readout_head(h, batch2d, p1, p2):
    W1, b1 = p1
    W2, b2 = p2

    def body(h_ref, b_ref, w1_ref, b1_ref, w2_ref, b2_ref, o_ref):
        oh = (b_ref[...] == lax.broadcasted_iota(jnp.int32, (_G, _N), 0)).astype(jnp.float32)
        g = jnp.dot(oh, h_ref[...], preferred_element_type=jnp.float32)
        a = jnp.maximum(
            jnp.dot(g, w1_ref[...], preferred_element_type=jnp.float32) + b1_ref[...], 0.0
        )
        o_ref[...] = jnp.dot(a, w2_ref[...], preferred_element_type=jnp.float32) + b2_ref[...]

    return pl.pallas_call(
        body, out_shape=jax.ShapeDtypeStruct((_G, 12), jnp.float32)
    )(h, batch2d, W1, b1.reshape(1, -1), W2, b2.reshape(1, -1))


def kernel(x, edge_index, batch, edge_attr, params):
    We, be = params["edge_emb"]
    Wn, bn = params["node_emb"]
    Wp = jnp.stack([We @ lp["lin_e"][0] for lp in params["layers"]])
    bp = jnp.stack(
        [(be @ lp["lin_e"][0] + lp["lin_e"][1]).reshape(1, _H) for lp in params["layers"]]
    )
    ea_all = _edge_linear(edge_attr, Wp, bp)
    src = edge_index[0]
    dst = edge_index[1]
    h = _node_embed(x, Wn, bn.reshape(1, _H))
    for l, lp in enumerate(params["layers"]):
        aggp = _sc_edge_pass(l)(h, ea_all, src, dst)
        h = _node_update(h, aggp, lp)
    return _readout_head(h, batch.reshape(1, _N), params["head1"], params["head2"])


# speedup vs baseline: 4.6235x; 1.0099x over previous
import functools

import jax
import jax.numpy as jnp
from jax import lax
from jax.experimental import pallas as pl
from jax.experimental.pallas import tpu as pltpu
from jax.experimental.pallas import tpu_sc as plsc

_N = 10000
_E = 320000
_H = 128
_G = 64
_L = 4

_NC = 2
_NS = 16
_NW = _NC * _NS
_EW = _E // _NW
_K = 80
_NCH = _EW // _K
_CZ = 80
_NQ = _N // _CZ
_NIB = 4


def _edge_linear(edge_attr, Wl, bl):
    BE = 2000

    def body(x_ref, w_ref, b_ref, o_ref):
        o_ref[...] = jnp.dot(x_ref[...], w_ref[...], preferred_element_type=jnp.float32) + b_ref[...]

    return pl.pallas_call(
        body,
        grid=(_E // BE,),
        in_specs=[
            pl.BlockSpec((BE, 16), lambda i: (i, 0)),
            pl.BlockSpec((16, _H), lambda i: (0, 0)),
            pl.BlockSpec((1, _H), lambda i: (0, 0)),
        ],
        out_specs=pl.BlockSpec((BE, _H), lambda i: (i, 0)),
        out_shape=jax.ShapeDtypeStruct((_E, _H), jnp.float32),
    )(edge_attr, Wl, bl)


def _node_embed(x, W, b):
    def body(x_ref, w_ref, b_ref, o_ref):
        o_ref[...] = jnp.dot(x_ref[...], w_ref[...], preferred_element_type=jnp.float32) + b_ref[...]

    return pl.pallas_call(
        body, out_shape=jax.ShapeDtypeStruct((_N, _H), jnp.float32)
    )(x, W, b)


def _make_sc_edge_pass():
    mesh = plsc.VectorSubcoreMesh(core_axis_name="c", subcore_axis_name="s")

    @functools.partial(
        pl.kernel,
        out_type=jax.ShapeDtypeStruct((_NC, _N, _H), jnp.float32),
        mesh=mesh,
        scratch_types=[
            pltpu.VMEM((2 * _NIB, _K), jnp.int32),
            pltpu.VMEM((2, _K, _H), jnp.float32),
            pltpu.VMEM((2, _K, _H), jnp.float32),
            pltpu.VMEM_SHARED((_N, _H), jnp.float32),
            pltpu.SemaphoreType.DMA((_NIB,)),
            pltpu.SemaphoreType.DMA((2,)),
            pltpu.SemaphoreType.DMA((2,)),
            pltpu.SemaphoreType.DMA((2,)),
        ],
    )
    def k(h_hbm, ea_hbm, src_hbm, dst_hbm, out_hbm, idxb, hbuf, eabuf,
          agg_sh, sem_i, sem_g, sem_e, sem_s):
        c = lax.axis_index("c")
        s = lax.axis_index("s")
        w = c * _NS + s

        zero16 = jnp.zeros((16,), jnp.float32)

        def zrow(e, carry):
            for f in range(_H // 16):
                hbuf[0, e, pl.ds(f * 16, 16)] = zero16
            return carry

        lax.fori_loop(0, _CZ, zrow, 0)
        nq = (_NQ - s + _NS - 1) // _NS

        def zchunk(t, carry):
            r0 = pl.multiple_of((s + t * _NS) * _CZ, 8)
            pltpu.sync_copy(hbuf.at[0], agg_sh.at[pl.ds(r0, _CZ)])
            return carry

        lax.fori_loop(0, nq, zchunk, 0)
        plsc.subcore_barrier()

        def issue_idx(j, m):
            e0 = pl.multiple_of(w * _EW + j * _K, 8)
            pltpu.async_copy(src_hbm.at[pl.ds(e0, _K)], idxb.at[2 * m], sem_i.at[m])
            pltpu.async_copy(dst_hbm.at[pl.ds(e0, _K)], idxb.at[2 * m + 1], sem_i.at[m])

        def wait_idx(j, m):
            e0 = pl.multiple_of(w * _EW + j * _K, 8)
            pltpu.make_async_copy(src_hbm.at[pl.ds(e0, _K)], idxb.at[2 * m], sem_i.at[m]).wait()
            pltpu.make_async_copy(dst_hbm.at[pl.ds(e0, _K)], idxb.at[2 * m + 1], sem_i.at[m]).wait()

        def issue_gather(m, b):
            pltpu.async_copy(h_hbm.at[idxb.at[2 * m]], hbuf.at[b], sem_g.at[b])

        def wait_gather(m, b):
            pltpu.make_async_copy(h_hbm.at[idxb.at[2 * m]], hbuf.at[b], sem_g.at[b]).wait()

        def issue_ea(j, b):
            e0 = pl.multiple_of(w * _EW + j * _K, 8)
            pltpu.async_copy(ea_hbm.at[pl.ds(e0, _K)], eabuf.at[b], sem_e.at[b])

        def wait_ea(j, b):
            e0 = pl.multiple_of(w * _EW + j * _K, 8)
            pltpu.make_async_copy(ea_hbm.at[pl.ds(e0, _K)], eabuf.at[b], sem_e.at[b]).wait()

        def issue_scatter(m, b):
            pltpu.async_copy(eabuf.at[b], agg_sh.at[idxb.at[2 * m + 1]], sem_s.at[b], add=True)

        def wait_scatter(m, b):
            pltpu.make_async_copy(eabuf.at[b], agg_sh.at[idxb.at[2 * m + 1]], sem_s.at[b]).wait()

        def compute(b):
            @plsc.parallel_loop(0, _K, 1, unroll=2)
            def row(e):
                for f in range(_H // 16):
                    sl = pl.ds(f * 16, 16)
                    eabuf[b, e, sl] = jnp.maximum(hbuf[b, e, sl] + eabuf[b, e, sl], 0.0)

        def body(j, m):
            b = m % 2

            wait_gather(m, b)
            wait_ea(j, b)

            @pl.when(j + 1 < _NCH)
            def _():
                wait_idx(j + 1, (m + 1) % _NIB)
                issue_gather((m + 1) % _NIB, 1 - b)

            @pl.when(j + 2 < _NCH)
            def _():
                issue_idx(j + 2, (m + 2) % _NIB)

            compute(b)

            @pl.when(j >= 1)
            def _():
                wait_scatter((m + _NIB - 1) % _NIB, 1 - b)

            @pl.when(j + 1 < _NCH)
            def _():
                issue_ea(j + 1, 1 - b)

            issue_scatter(m, b)

        issue_idx(0, 0)
        issue_idx(1, 1)
        wait_idx(0, 0)
        issue_gather(0, 0)
        issue_ea(0, 0)

        def quad(q, carry):
            for m in range(_NIB):
                body(_NIB * q + m, m)
            return carry

        lax.fori_loop(0, _NCH // _NIB, quad, 0)
        body(_NCH - 1, 0)
        wait_scatter(0, 0)
        plsc.subcore_barrier()

        def ochunk(t, carry):
            r0 = pl.multiple_of((s + t * _NS) * _CZ, 8)
            pltpu.sync_copy(agg_sh.at[pl.ds(r0, _CZ)], hbuf.at[0])
            pltpu.sync_copy(hbuf.at[0], out_hbm.at[c, pl.ds(r0, _CZ)])
            return carry

        lax.fori_loop(0, nq, ochunk, 0)

    return k


_SC_EDGE_PASS = _make_sc_edge_pass()


def _node_update(h, aggp, lp):
    W1, b1 = lp["mlp1"]
    W2, b2 = lp["mlp2"]
    gamma, beta = lp["bn"]

    def body(h_ref, a_ref, w1_ref, b1_ref, w2_ref, b2_ref, g_ref, bb_ref, o_ref):
        z = h_ref[...] + a_ref[0] + a_ref[1]
        a1 = jnp.maximum(
            jnp.dot(z, w1_ref[...], preferred_element_type=jnp.float32) + b1_ref[...], 0.0
        )
        out = jnp.dot(a1, w2_ref[...], preferred_element_type=jnp.float32) + b2_ref[...]
        mean = jnp.mean(out, axis=0, keepdims=True)
        d = out - mean
        var = jnp.mean(d * d, axis=0, keepdims=True)
        o_ref[...] = jnp.maximum(
            d * lax.rsqrt(var + 1e-5) * g_ref[...] + bb_ref[...], 0.0
        )

    return pl.pallas_call(
        body, out_shape=jax.ShapeDtypeStruct((_N, _H), jnp.float32)
    )(
        h, aggp, W1, b1.reshape(1, _H), W2, b2.reshape(1, _H),
        gamma.reshape(1, _H), beta.reshape(1, _H),
    )


def _readout_head(h, batch2d, p1, p2):
    W1, b1 = p1
    W2, b2 = p2

    def body(h_ref, b_ref, w1_ref, b1_ref, w2_ref, b2_ref, o_ref):
        oh = (b_ref[...] == lax.broadcasted_iota(jnp.int32, (_G, _N), 0)).astype(jnp.float32)
        g = jnp.dot(oh, h_ref[...], preferred_element_type=jnp.float32)
        a = jnp.maximum(
            jnp.dot(g, w1_ref[...], preferred_element_type=jnp.float32) + b1_ref[...], 0.0
        )
        o_ref[...] = jnp.dot(a, w2_ref[...], preferred_element_type=jnp.float32) + b2_ref[...]

    return pl.pallas_call(
        body, out_shape=jax.ShapeDtypeStruct((_G, 12), jnp.float32)
    )(h, batch2d, W1, b1.reshape(1, -1), W2, b2.reshape(1, -1))


def kernel(x, edge_index, batch, edge_attr, params):
    We, be = params["edge_emb"]
    Wn, bn = params["node_emb"]
    src = edge_index[0]
    dst = edge_index[1]
    h = _node_embed(x, Wn, bn.reshape(1, _H))
    for lp in params["layers"]:
        Wl = We @ lp["lin_e"][0]
        bl = (be @ lp["lin_e"][0] + lp["lin_e"][1]).reshape(1, _H)
        ea_l = _edge_linear(edge_attr, Wl, bl)
        aggp = _SC_EDGE_PASS(h, ea_l, src, dst)
        h = _node_update(h, aggp, lp)
    return _readout_head(h, batch.reshape(1, _N), params["head1"], params["head2"])


# speedup vs baseline: 4.6379x; 1.0031x over previous
import functools

import jax
import jax.numpy as jnp
from jax import lax
from jax.experimental import pallas as pl
from jax.experimental.pallas import tpu as pltpu
from jax.experimental.pallas import tpu_sc as plsc

_N = 10000
_E = 320000
_H = 128
_G = 64
_L = 4

_NC = 2
_NS = 16
_NW = _NC * _NS
_EW = _E // _NW
_K = 80
_NCH = _EW // _K
_CZ = 80
_NQ = _N // _CZ
_NIB = 4


def _edge_linear(edge_attr, Wl, bl):
    BE = 2000

    def body(x_ref, w_ref, b_ref, o_ref):
        o_ref[...] = jnp.dot(x_ref[...], w_ref[...], preferred_element_type=jnp.float32) + b_ref[...]

    return pl.pallas_call(
        body,
        grid=(_E // BE,),
        in_specs=[
            pl.BlockSpec((BE, 16), lambda i: (i, 0)),
            pl.BlockSpec((16, _H), lambda i: (0, 0)),
            pl.BlockSpec((1, _H), lambda i: (0, 0)),
        ],
        out_specs=pl.BlockSpec((BE, _H), lambda i: (i, 0)),
        out_shape=jax.ShapeDtypeStruct((_E, _H), jnp.float32),
    )(edge_attr, Wl, bl)


def _node_embed(x, W, b):
    def body(x_ref, w_ref, b_ref, o_ref):
        o_ref[...] = jnp.dot(x_ref[...], w_ref[...], preferred_element_type=jnp.float32) + b_ref[...]

    return pl.pallas_call(
        body, out_shape=jax.ShapeDtypeStruct((_N, _H), jnp.float32)
    )(x, W, b)


def _make_sc_edge_pass():
    mesh = plsc.VectorSubcoreMesh(core_axis_name="c", subcore_axis_name="s")

    @functools.partial(
        pl.kernel,
        out_type=jax.ShapeDtypeStruct((_NC, _N, _H), jnp.float32),
        mesh=mesh,
        scratch_types=[
            pltpu.VMEM((2 * _NIB, _K), jnp.int32),
            pltpu.VMEM((2, _K, _H), jnp.float32),
            pltpu.VMEM((2, _K, _H), jnp.float32),
            pltpu.VMEM_SHARED((_N, _H), jnp.float32),
            pltpu.SemaphoreType.DMA((_NIB,)),
            pltpu.SemaphoreType.DMA((2,)),
            pltpu.SemaphoreType.DMA((2,)),
            pltpu.SemaphoreType.DMA((2,)),
        ],
    )
    def k(h_hbm, ea_hbm, src_hbm, dst_hbm, out_hbm, idxb, hbuf, eabuf,
          agg_sh, sem_i, sem_g, sem_e, sem_s):
        c = lax.axis_index("c")
        s = lax.axis_index("s")
        w = c * _NS + s

        zero16 = jnp.zeros((16,), jnp.float32)

        def zrow(e, carry):
            for f in range(_H // 16):
                hbuf[0, e, pl.ds(f * 16, 16)] = zero16
            return carry

        lax.fori_loop(0, _CZ, zrow, 0)
        nq = (_NQ - s + _NS - 1) // _NS

        def zchunk(t, carry):
            r0 = pl.multiple_of((s + t * _NS) * _CZ, 8)
            pltpu.sync_copy(hbuf.at[0], agg_sh.at[pl.ds(r0, _CZ)])
            return carry

        lax.fori_loop(0, nq, zchunk, 0)
        plsc.subcore_barrier()

        def issue_idx(j, m):
            e0 = pl.multiple_of(w * _EW + j * _K, 8)
            pltpu.async_copy(src_hbm.at[pl.ds(e0, _K)], idxb.at[2 * m], sem_i.at[m])
            pltpu.async_copy(dst_hbm.at[pl.ds(e0, _K)], idxb.at[2 * m + 1], sem_i.at[m])

        def wait_idx(j, m):
            e0 = pl.multiple_of(w * _EW + j * _K, 8)
            pltpu.make_async_copy(src_hbm.at[pl.ds(e0, _K)], idxb.at[2 * m], sem_i.at[m]).wait()
            pltpu.make_async_copy(dst_hbm.at[pl.ds(e0, _K)], idxb.at[2 * m + 1], sem_i.at[m]).wait()

        def issue_gather(m, b):
            pltpu.async_copy(h_hbm.at[idxb.at[2 * m]], hbuf.at[b], sem_g.at[b])

        def wait_gather(m, b):
            pltpu.make_async_copy(h_hbm.at[idxb.at[2 * m]], hbuf.at[b], sem_g.at[b]).wait()

        def issue_ea(j, b):
            e0 = pl.multiple_of(w * _EW + j * _K, 8)
            pltpu.async_copy(ea_hbm.at[pl.ds(e0, _K)], eabuf.at[b], sem_e.at[b])

        def wait_ea(j, b):
            e0 = pl.multiple_of(w * _EW + j * _K, 8)
            pltpu.make_async_copy(ea_hbm.at[pl.ds(e0, _K)], eabuf.at[b], sem_e.at[b]).wait()

        def issue_scatter(m, b):
            pltpu.async_copy(eabuf.at[b], agg_sh.at[idxb.at[2 * m + 1]], sem_s.at[b], add=True)

        def wait_scatter(m, b):
            pltpu.make_async_copy(eabuf.at[b], agg_sh.at[idxb.at[2 * m + 1]], sem_s.at[b]).wait()

        def compute(b):
            @plsc.parallel_loop(0, _K, 1, unroll=4)
            def row(e):
                for f in range(_H // 16):
                    sl = pl.ds(f * 16, 16)
                    eabuf[b, e, sl] = jnp.maximum(hbuf[b, e, sl] + eabuf[b, e, sl], 0.0)

        def body(j, m):
            b = m % 2

            wait_gather(m, b)
            wait_ea(j, b)

            @pl.when(j + 1 < _NCH)
            def _():
                wait_idx(j + 1, (m + 1) % _NIB)
                issue_gather((m + 1) % _NIB, 1 - b)

            @pl.when(j + 2 < _NCH)
            def _():
                issue_idx(j + 2, (m + 2) % _NIB)

            compute(b)

            @pl.when(j >= 1)
            def _():
                wait_scatter((m + _NIB - 1) % _NIB, 1 - b)

            @pl.when(j + 1 < _NCH)
            def _():
                issue_ea(j + 1, 1 - b)

            issue_scatter(m, b)

        issue_idx(0, 0)
        issue_idx(1, 1)
        wait_idx(0, 0)
        issue_gather(0, 0)
        issue_ea(0, 0)

        def quad(q, carry):
            for m in range(_NIB):
                body(_NIB * q + m, m)
            return carry

        lax.fori_loop(0, _NCH // _NIB, quad, 0)
        body(_NCH - 1, 0)
        wait_scatter(0, 0)
        plsc.subcore_barrier()

        def ochunk(t, carry):
            r0 = pl.multiple_of((s + t * _NS) * _CZ, 8)
            pltpu.sync_copy(agg_sh.at[pl.ds(r0, _CZ)], hbuf.at[0])
            pltpu.sync_copy(hbuf.at[0], out_hbm.at[c, pl.ds(r0, _CZ)])
            return carry

        lax.fori_loop(0, nq, ochunk, 0)

    return k


_SC_EDGE_PASS = _make_sc_edge_pass()


def _node_update(h, aggp, lp):
    W1, b1 = lp["mlp1"]
    W2, b2 = lp["mlp2"]
    gamma, beta = lp["bn"]

    def body(h_ref, a_ref, w1_ref, b1_ref, w2_ref, b2_ref, g_ref, bb_ref, o_ref):
        z = h_ref[...] + a_ref[0] + a_ref[1]
        a1 = jnp.maximum(
            jnp.dot(z, w1_ref[...], preferred_element_type=jnp.float32) + b1_ref[...], 0.0
        )
        out = jnp.dot(a1, w2_ref[...], preferred_element_type=jnp.float32) + b2_ref[...]
        mean = jnp.mean(out, axis=0, keepdims=True)
        d = out - mean
        var = jnp.mean(d * d, axis=0, keepdims=True)
        o_ref[...] = jnp.maximum(
            d * lax.rsqrt(var + 1e-5) * g_ref[...] + bb_ref[...], 0.0
        )

    return pl.pallas_call(
        body, out_shape=jax.ShapeDtypeStruct((_N, _H), jnp.float32)
    )(
        h, aggp, W1, b1.reshape(1, _H), W2, b2.reshape(1, _H),
        gamma.reshape(1, _H), beta.reshape(1, _H),
    )


def _readout_head(h, batch2d, p1, p2):
    W1, b1 = p1
    W2, b2 = p2

    def body(h_ref, b_ref, w1_ref, b1_ref, w2_ref, b2_ref, o_ref):
        oh = (b_ref[...] == lax.broadcasted_iota(jnp.int32, (_G, _N), 0)).astype(jnp.float32)
        g = jnp.dot(oh, h_ref[...], preferred_element_type=jnp.float32)
        a = jnp.maximum(
            jnp.dot(g, w1_ref[...], preferred_element_type=jnp.float32) + b1_ref[...], 0.0
        )
        o_ref[...] = jnp.dot(a, w2_ref[...], preferred_element_type=jnp.float32) + b2_ref[...]

    return pl.pallas_call(
        body, out_shape=jax.ShapeDtypeStruct((_G, 12), jnp.float32)
    )(h, batch2d, W1, b1.reshape(1, -1), W2, b2.reshape(1, -1))


def kernel(x, edge_index, batch, edge_attr, params):
    We, be = params["edge_emb"]
    Wn, bn = params["node_emb"]
    src = edge_index[0]
    dst = edge_index[1]
    h = _node_embed(x, Wn, bn.reshape(1, _H))
    for lp in params["layers"]:
        Wl = We @ lp["lin_e"][0]
        bl = (be @ lp["lin_e"][0] + lp["lin_e"][1]).reshape(1, _H)
        ea_l = _edge_linear(edge_attr, Wl, bl)
        aggp = _SC_EDGE_PASS(h, ea_l, src, dst)
        h = _node_update(h, aggp, lp)
    return _readout_head(h, batch.reshape(1, _N), params["head1"], params["head2"])


# speedup vs baseline: 4.6769x; 1.0084x over previous
import functools

import jax
import jax.numpy as jnp
from jax import lax
from jax.experimental import pallas as pl
from jax.experimental.pallas import tpu as pltpu
from jax.experimental.pallas import tpu_sc as plsc

_N = 10000
_E = 320000
_H = 128
_G = 64
_L = 4

_NC = 2
_NS = 16
_NW = _NC * _NS
_EW = _E // _NW
_K = 80
_NCH = _EW // _K
_CZ = 80
_NQ = _N // _CZ
_NIB = 4


def _edge_linear(edge_attr, Wl, bl):
    BE = 2000

    def body(x_ref, w_ref, b_ref, o_ref):
        o_ref[...] = jnp.dot(x_ref[...], w_ref[...], preferred_element_type=jnp.float32) + b_ref[...]

    return pl.pallas_call(
        body,
        grid=(_E // BE,),
        in_specs=[
            pl.BlockSpec((BE, 16), lambda i: (i, 0)),
            pl.BlockSpec((16, _H), lambda i: (0, 0)),
            pl.BlockSpec((1, _H), lambda i: (0, 0)),
        ],
        out_specs=pl.BlockSpec((BE, _H), lambda i: (i, 0)),
        out_shape=jax.ShapeDtypeStruct((_E, _H), jnp.float32),
    )(edge_attr, Wl, bl)


def _node_embed(x, W, b):
    def body(x_ref, w_ref, b_ref, o_ref):
        o_ref[...] = jnp.dot(x_ref[...], w_ref[...], preferred_element_type=jnp.float32) + b_ref[...]

    return pl.pallas_call(
        body, out_shape=jax.ShapeDtypeStruct((_N, _H), jnp.float32)
    )(x, W, b)


def _make_sc_edge_pass():
    mesh = plsc.VectorSubcoreMesh(core_axis_name="c", subcore_axis_name="s")

    @functools.partial(
        pl.kernel,
        out_type=jax.ShapeDtypeStruct((_NC, _N, _H), jnp.float32),
        mesh=mesh,
        scratch_types=[
            pltpu.VMEM((2 * _NIB, _K), jnp.int32),
            pltpu.VMEM((2, _K, _H), jnp.float32),
            pltpu.VMEM((2, _K, _H), jnp.float32),
            pltpu.VMEM_SHARED((_N, _H), jnp.float32),
            pltpu.SemaphoreType.DMA((_NIB,)),
            pltpu.SemaphoreType.DMA((2,)),
            pltpu.SemaphoreType.DMA((2,)),
            pltpu.SemaphoreType.DMA((2,)),
        ],
    )
    def k(h_hbm, ea_hbm, src_hbm, dst_hbm, out_hbm, idxb, hbuf, eabuf,
          agg_sh, sem_i, sem_g, sem_e, sem_s):
        c = lax.axis_index("c")
        s = lax.axis_index("s")
        w = c * _NS + s

        zero16 = jnp.zeros((16,), jnp.float32)

        def zrow(e, carry):
            for f in range(_H // 16):
                hbuf[0, e, pl.ds(f * 16, 16)] = zero16
            return carry

        lax.fori_loop(0, _CZ, zrow, 0)
        nq = (_NQ - s + _NS - 1) // _NS

        def zchunk(t, carry):
            r0 = pl.multiple_of((s + t * _NS) * _CZ, 8)
            pltpu.async_copy(hbuf.at[0], agg_sh.at[pl.ds(r0, _CZ)], sem_g.at[0])
            return carry

        lax.fori_loop(0, nq, zchunk, 0)

        def zdrain(t, carry):
            r0 = pl.multiple_of((s + t * _NS) * _CZ, 8)
            pltpu.make_async_copy(hbuf.at[0], agg_sh.at[pl.ds(r0, _CZ)], sem_g.at[0]).wait()
            return carry

        lax.fori_loop(0, nq, zdrain, 0)
        plsc.subcore_barrier()

        def issue_idx(j, m):
            e0 = pl.multiple_of(w * _EW + j * _K, 8)
            pltpu.async_copy(src_hbm.at[pl.ds(e0, _K)], idxb.at[2 * m], sem_i.at[m])
            pltpu.async_copy(dst_hbm.at[pl.ds(e0, _K)], idxb.at[2 * m + 1], sem_i.at[m])

        def wait_idx(j, m):
            e0 = pl.multiple_of(w * _EW + j * _K, 8)
            pltpu.make_async_copy(src_hbm.at[pl.ds(e0, _K)], idxb.at[2 * m], sem_i.at[m]).wait()
            pltpu.make_async_copy(dst_hbm.at[pl.ds(e0, _K)], idxb.at[2 * m + 1], sem_i.at[m]).wait()

        def issue_gather(m, b):
            pltpu.async_copy(h_hbm.at[idxb.at[2 * m]], hbuf.at[b], sem_g.at[b])

        def wait_gather(m, b):
            pltpu.make_async_copy(h_hbm.at[idxb.at[2 * m]], hbuf.at[b], sem_g.at[b]).wait()

        def issue_ea(j, b):
            e0 = pl.multiple_of(w * _EW + j * _K, 8)
            pltpu.async_copy(ea_hbm.at[pl.ds(e0, _K)], eabuf.at[b], sem_e.at[b])

        def wait_ea(j, b):
            e0 = pl.multiple_of(w * _EW + j * _K, 8)
            pltpu.make_async_copy(ea_hbm.at[pl.ds(e0, _K)], eabuf.at[b], sem_e.at[b]).wait()

        def issue_scatter(m, b):
            pltpu.async_copy(eabuf.at[b], agg_sh.at[idxb.at[2 * m + 1]], sem_s.at[b], add=True)

        def wait_scatter(m, b):
            pltpu.make_async_copy(eabuf.at[b], agg_sh.at[idxb.at[2 * m + 1]], sem_s.at[b]).wait()

        def compute(b):
            @plsc.parallel_loop(0, _K, 1, unroll=4)
            def row(e):
                for f in range(_H // 16):
                    sl = pl.ds(f * 16, 16)
                    eabuf[b, e, sl] = jnp.maximum(hbuf[b, e, sl] + eabuf[b, e, sl], 0.0)

        def body(j, m):
            b = m % 2

            wait_gather(m, b)
            wait_ea(j, b)

            @pl.when(j + 1 < _NCH)
            def _():
                wait_idx(j + 1, (m + 1) % _NIB)
                issue_gather((m + 1) % _NIB, 1 - b)

            @pl.when(j + 2 < _NCH)
            def _():
                issue_idx(j + 2, (m + 2) % _NIB)

            compute(b)

            @pl.when(j >= 1)
            def _():
                wait_scatter((m + _NIB - 1) % _NIB, 1 - b)

            @pl.when(j + 1 < _NCH)
            def _():
                issue_ea(j + 1, 1 - b)

            issue_scatter(m, b)

        issue_idx(0, 0)
        issue_idx(1, 1)
        wait_idx(0, 0)
        issue_gather(0, 0)
        issue_ea(0, 0)

        def quad(q, carry):
            for m in range(_NIB):
                body(_NIB * q + m, m)
            return carry

        lax.fori_loop(0, _NCH // _NIB, quad, 0)
        body(_NCH - 1, 0)
        wait_scatter(0, 0)
        plsc.subcore_barrier()

        def orow(t):
            return pl.multiple_of((s + t * _NS) * _CZ, 8)

        def oin(t, b):
            pltpu.async_copy(agg_sh.at[pl.ds(orow(t), _CZ)], hbuf.at[b], sem_g.at[b])

        def oin_wait(t, b):
            pltpu.make_async_copy(agg_sh.at[pl.ds(orow(t), _CZ)], hbuf.at[b], sem_g.at[b]).wait()

        def oout(t, b):
            pltpu.async_copy(hbuf.at[b], out_hbm.at[c, pl.ds(orow(t), _CZ)], sem_s.at[b])

        def oout_wait(t, b):
            pltpu.make_async_copy(hbuf.at[b], out_hbm.at[c, pl.ds(orow(t), _CZ)], sem_s.at[b]).wait()

        @pl.when(nq >= 1)
        def _():
            oin(0, 0)

        def opair(t, carry):
            for b in range(2):
                tt = 2 * t + b

                @pl.when(tt < nq)
                def _():
                    oin_wait(tt, b)

                    @pl.when(tt >= 1)
                    def _():
                        oout_wait(tt - 1, 1 - b)

                    @pl.when(tt + 1 < nq)
                    def _():
                        oin(tt + 1, 1 - b)

                    oout(tt, b)
            return carry

        lax.fori_loop(0, (_NQ // _NS + 2 + 1) // 2, opair, 0)

        par = (nq - 1) % 2

        @pl.when((nq >= 1) & (par == 0))
        def _():
            oout_wait(nq - 1, 0)

        @pl.when((nq >= 1) & (par == 1))
        def _():
            oout_wait(nq - 1, 1)

    return k


_SC_EDGE_PASS = _make_sc_edge_pass()


def _node_update(h, aggp, lp):
    W1, b1 = lp["mlp1"]
    W2, b2 = lp["mlp2"]
    gamma, beta = lp["bn"]

    def body(h_ref, a_ref, w1_ref, b1_ref, w2_ref, b2_ref, g_ref, bb_ref, o_ref):
        z = h_ref[...] + a_ref[0] + a_ref[1]
        a1 = jnp.maximum(
            jnp.dot(z, w1_ref[...], preferred_element_type=jnp.float32) + b1_ref[...], 0.0
        )
        out = jnp.dot(a1, w2_ref[...], preferred_element_type=jnp.float32) + b2_ref[...]
        mean = jnp.mean(out, axis=0, keepdims=True)
        d = out - mean
        var = jnp.mean(d * d, axis=0, keepdims=True)
        o_ref[...] = jnp.maximum(
            d * lax.rsqrt(var + 1e-5) * g_ref[...] + bb_ref[...], 0.0
        )

    return pl.pallas_call(
        body, out_shape=jax.ShapeDtypeStruct((_N, _H), jnp.float32)
    )(
        h, aggp, W1, b1.reshape(1, _H), W2, b2.reshape(1, _H),
        gamma.reshape(1, _H), beta.reshape(1, _H),
    )


def _readout_head(h, batch2d, p1, p2):
    W1, b1 = p1
    W2, b2 = p2

    def body(h_ref, b_ref, w1_ref, b1_ref, w2_ref, b2_ref, o_ref):
        oh = (b_ref[...] == lax.broadcasted_iota(jnp.int32, (_G, _N), 0)).astype(jnp.float32)
        g = jnp.dot(oh, h_ref[...], preferred_element_type=jnp.float32)
        a = jnp.maximum(
            jnp.dot(g, w1_ref[...], preferred_element_type=jnp.float32) + b1_ref[...], 0.0
        )
        o_ref[...] = jnp.dot(a, w2_ref[...], preferred_element_type=jnp.float32) + b2_ref[...]

    return pl.pallas_call(
        body, out_shape=jax.ShapeDtypeStruct((_G, 12), jnp.float32)
    )(h, batch2d, W1, b1.reshape(1, -1), W2, b2.reshape(1, -1))


def kernel(x, edge_index, batch, edge_attr, params):
    We, be = params["edge_emb"]
    Wn, bn = params["node_emb"]
    src = edge_index[0]
    dst = edge_index[1]
    h = _node_embed(x, Wn, bn.reshape(1, _H))
    for lp in params["layers"]:
        Wl = We @ lp["lin_e"][0]
        bl = (be @ lp["lin_e"][0] + lp["lin_e"][1]).reshape(1, _H)
        ea_l = _edge_linear(edge_attr, Wl, bl)
        aggp = _SC_EDGE_PASS(h, ea_l, src, dst)
        h = _node_update(h, aggp, lp)
    return _readout_head(h, batch.reshape(1, _N), params["head1"], params["head2"])
